# Initial kernel scaffold; baseline (speedup 1.0000x reference)
import jax
import jax.numpy as jnp
from jax.experimental import pallas as pl


def kernel(xyz, Wp, Ws, bs, W1, b1, W2, b2, Wa1, ba1, Wa2, ba2, Wm1, bm1, Wm2, bm2):
    raise NotImplementedError("write your pallas kernel here")


# speedup vs baseline: 13.3063x; 13.3063x over previous
import jax
import jax.numpy as jnp
from jax.experimental import pallas as pl

_T = 256
_K = 16


def _body(xyz_ref, xyzT_ref, W1T, b1r, W2T, b2r, WpT, WsT, bsr,
          Wa1T, ba1r, Wa2T, ba2r, Wm1T, bm1r, Wm2T, bm2r, out_ref):
    N = xyz_ref.shape[2]
    i = pl.program_id(1)
    f32 = jnp.float32

    X = xyz_ref[0]
    G = xyzT_ref[0]
    Gt = xyzT_ref[0, pl.ds(i * _T, _T), :]

    def mm(a, w):
        return jax.lax.dot_general(a, w, (((1,), (0,)), ((), ())),
                                   preferred_element_type=f32)

    F = jax.nn.relu(mm(jax.nn.relu(mm(G, W1T[...]) + b1r[...]), W2T[...]) + b2r[...])
    ft = jax.nn.relu(mm(jax.nn.relu(mm(Gt, W1T[...]) + b1r[...]), W2T[...]) + b2r[...])
    FG = jnp.concatenate([F, G], axis=1)

    sq_row = jnp.sum(X * X, axis=0, keepdims=True)
    sq_col = jnp.sum(Gt * Gt, axis=1, keepdims=True)
    d2 = sq_col + sq_row - 2.0 * mm(Gt, X)

    iota = jax.lax.broadcasted_iota(jnp.int32, (_T, N), 1)

    s_list = []
    fk_list = []
    planes = None
    for k in range(_K):
        rowmin = jnp.min(d2, axis=1, keepdims=True)
        seleq = d2 == rowmin
        idxv = jnp.min(jnp.where(seleq, iota, N), axis=1, keepdims=True)
        sel = iota == idxv
        onehot = sel.astype(f32)
        d2 = jnp.where(sel, jnp.float32(jnp.inf), d2)
        gp = mm(onehot, FG)
        gk = gp[:, :64]
        fk = gk - ft
        fk_list.append(fk)
        a = jax.nn.relu(mm(fk, Wa1T[...]) + ba1r[...])
        s_list.append(mm(a, Wa2T[...]) + ba2r[...])
        if k > 0:
            dk = gp[:, 64:67] - Gt
            nrm = jnp.sqrt(jnp.sum(dk * dk, axis=1, keepdims=True)) + 1e-8
            plk = mm(dk, WpT[...]) / nrm
            contrib = nrm * plk * jnp.abs(plk)
            planes = contrib if planes is None else jnp.maximum(planes, contrib)

    m = s_list[0]
    for s in s_list[1:]:
        m = jnp.maximum(m, s)
    z = jnp.zeros_like(m)
    acc = jnp.zeros_like(fk_list[0])
    for s, fk in zip(s_list, fk_list):
        e = jnp.exp(s - m)
        z = z + e
        acc = acc + e * fk
    f_att = acc / z

    f_shapes = mm(planes, WsT[...]) + bsr[...]
    fc = jnp.concatenate([f_att, f_shapes], axis=1)
    h = jax.nn.relu(mm(fc, Wm1T[...]) + bm1r[...])
    o3 = mm(h, Wm2T[...]) + bm2r[...]

    xyz6 = jnp.concatenate([Gt[:, 0:1], Gt[:, 0:1],
                            Gt[:, 1:2], Gt[:, 1:2],
                            Gt[:, 2:3], Gt[:, 2:3]], axis=1)
    out_ref[0] = xyz6 + 0.15 * o3


def kernel(xyz, Wp, Ws, bs, W1, b1, W2, b2, Wa1, ba1, Wa2, ba2, Wm1, bm1, Wm2, bm2):
    B, C, N = xyz.shape
    f32 = jnp.float32
    xyz = xyz.astype(f32)
    xyzT = jnp.transpose(xyz, (0, 2, 1))

    def full2d(a):
        return pl.BlockSpec(a.shape, lambda b, t: (0, 0))

    ins = [
        xyz, xyzT,
        W1.T, b1.reshape(1, -1), W2.T, b2.reshape(1, -1),
        Wp.T, Ws.T, bs.reshape(1, -1),
        Wa1.T, ba1.reshape(1, -1), Wa2.T, ba2.reshape(1, -1),
        Wm1.T, bm1.reshape(1, -1), Wm2.T, bm2.reshape(1, -1),
    ]
    in_specs = [
        pl.BlockSpec((1, C, N), lambda b, t: (b, 0, 0)),
        pl.BlockSpec((1, N, C), lambda b, t: (b, 0, 0)),
    ] + [full2d(a) for a in ins[2:]]

    y = pl.pallas_call(
        _body,
        grid=(B, N // _T),
        in_specs=in_specs,
        out_specs=pl.BlockSpec((1, _T, 6), lambda b, t: (b, t, 0)),
        out_shape=jax.ShapeDtypeStruct((B, N, 6), f32),
    )(*ins)

    return y.transpose(0, 2, 1).reshape(B, 3, 2 * N)
